# baseline (speedup 1.0000x reference)
import functools

import jax
import jax.numpy as jnp
from jax import lax
from jax.experimental import pallas as pl
from jax.experimental.pallas import tpu as pltpu
from jax.experimental.pallas import tpu_sc as plsc

_NC = 2
_NS = 16
_NW = _NC * _NS
_L = 16
_TW, _TD = 8, 128


@functools.lru_cache(maxsize=None)
def _build_sc_call(B, H, W, D):
    D2 = 2 * D
    PLANE = W * D2
    WT, DT = W // _TW, D2 // _TD
    DHALF = D // _TD
    assert H == _NW and D % _TD == 0 and W % _TW == 0

    mesh = plsc.VectorSubcoreMesh(core_axis_name="c", subcore_axis_name="s")

    @functools.partial(
        pl.kernel,
        out_type=jax.ShapeDtypeStruct((B * H * PLANE,), jnp.float32),
        mesh=mesh,
        scratch_types=[
            pltpu.VMEM((W * D,), jnp.float32),
            pltpu.VMEM((D,), jnp.float32),
            pltpu.VMEM((PLANE,), jnp.float32),
            pltpu.SemaphoreType.DMA,
        ],
        compiler_params=pltpu.CompilerParams(needs_layout_passes=False),
    )
    def sc_call(col_hbm, row_hbm, out_hbm, col_v, row_v, plane_v, sem):
        cid = lax.axis_index("c")
        sid = lax.axis_index("s")
        h = sid * _NC + cid
        ht = h // _TW
        hi = h - ht * _TW
        pltpu.sync_copy(col_hbm, col_v)
        for dt2 in range(D // _TD):
            roff = pl.multiple_of((ht * DHALF + dt2) * _TW * _TD + hi * _TD, _TD)
            pltpu.sync_copy(
                row_hbm.at[pl.ds(roff, _TD)], row_v.at[pl.ds(dt2 * _TD, _TD)]
            )

        rvecs = [row_v[pl.ds(t * _L, _L)] for t in range(D // _L)]

        def w_body(i, carry):
            wt = i // _TW
            wi = i - wt * _TW
            src0 = pl.multiple_of((wt * DHALF) * _TW * _TD + wi * _TD, _TD)
            dst0 = pl.multiple_of((wt * DT) * _TW * _TD + wi * _TD, _TD)
            for dt in range(DT):
                dst = dst0 + dt * _TW * _TD
                if dt < DHALF:
                    src = src0 + dt * _TW * _TD
                    for t in range(_TD // _L):
                        plane_v[pl.ds(dst + t * _L, _L)] = col_v[
                            pl.ds(src + t * _L, _L)
                        ]
                else:
                    for t in range(_TD // _L):
                        plane_v[pl.ds(dst + t * _L, _L)] = rvecs[
                            (dt - DHALF) * (_TD // _L) + t
                        ]
            return carry

        NQ = 4
        QUART = PLANE // NQ
        copies = []
        for q in range(NQ):
            lax.fori_loop(q * (W // NQ), (q + 1) * (W // NQ), w_body, 0)
            for b in range(B):
                off = pl.multiple_of((b * H + h) * PLANE + q * QUART, QUART)
                cp = pltpu.make_async_copy(
                    plane_v.at[pl.ds(q * QUART, QUART)],
                    out_hbm.at[pl.ds(off, QUART)],
                    sem,
                )
                cp.start()
                copies.append(cp)
        for cp in copies:
            cp.wait()

    return sc_call


def kernel(pixel_values, row_embeddings, column_embeddings):
    B = pixel_values.shape[0]
    H, W = pixel_values.shape[-2], pixel_values.shape[-1]
    D = row_embeddings.shape[-1]
    D2 = 2 * D
    col = (
        column_embeddings[:W]
        .reshape(W // _TW, _TW, D // _TD, _TD)
        .transpose(0, 2, 1, 3)
        .reshape(-1)
    )
    row = (
        row_embeddings[:H]
        .reshape(H // _TW, _TW, D // _TD, _TD)
        .transpose(0, 2, 1, 3)
        .reshape(-1)
    )
    out = _build_sc_call(B, H, W, D)(col, row)
    out6 = out.reshape(B, H, W // _TW, D2 // _TD, _TW, _TD)
    return jnp.transpose(out6, (0, 3, 5, 1, 2, 4)).reshape(B, D2, H, W)


# speedup vs baseline: 1.0228x; 1.0228x over previous
import functools

import jax
import jax.numpy as jnp
from jax import lax
from jax.experimental import pallas as pl
from jax.experimental.pallas import tpu as pltpu
from jax.experimental.pallas import tpu_sc as plsc

_NC = 2
_NS = 16
_NW = _NC * _NS
_L = 16
_TW, _TD = 8, 128


@functools.lru_cache(maxsize=None)
def _build_sc_call(B, H, W, D):
    D2 = 2 * D
    PLANE = W * D2
    WT, DT = W // _TW, D2 // _TD
    DHALF = D // _TD
    assert H == _NW and D % _TD == 0 and W % _TW == 0

    mesh = plsc.VectorSubcoreMesh(core_axis_name="c", subcore_axis_name="s")

    @functools.partial(
        pl.kernel,
        out_type=jax.ShapeDtypeStruct((B * H * PLANE,), jnp.float32),
        mesh=mesh,
        scratch_types=[
            pltpu.VMEM((W * D,), jnp.float32),
            pltpu.VMEM((D,), jnp.float32),
            pltpu.VMEM((PLANE,), jnp.float32),
            pltpu.SemaphoreType.DMA,
        ],
        compiler_params=pltpu.CompilerParams(needs_layout_passes=False),
    )
    def sc_call(col_hbm, row_hbm, out_hbm, col_v, row_v, plane_v, sem):
        cid = lax.axis_index("c")
        sid = lax.axis_index("s")
        h = sid * _NC + cid
        ht = h // _TW
        hi = h - ht * _TW
        pltpu.sync_copy(col_hbm, col_v)
        for dt2 in range(D // _TD):
            roff = pl.multiple_of((ht * DHALF + dt2) * _TW * _TD + hi * _TD, _TD)
            pltpu.sync_copy(
                row_hbm.at[pl.ds(roff, _TD)], row_v.at[pl.ds(dt2 * _TD, _TD)]
            )

        rvecs = [row_v[pl.ds(t * _L, _L)] for t in range(D // _L)]

        def w_body(i, carry):
            wt = i // _TW
            wi = i - wt * _TW
            src0 = pl.multiple_of((wt * DHALF) * _TW * _TD + wi * _TD, _TD)
            dst0 = pl.multiple_of((wt * DT) * _TW * _TD + wi * _TD, _TD)
            for dt in range(DT):
                dst = dst0 + dt * _TW * _TD
                if dt < DHALF:
                    src = src0 + dt * _TW * _TD
                    for t in range(_TD // _L):
                        plane_v[pl.ds(dst + t * _L, _L)] = col_v[
                            pl.ds(src + t * _L, _L)
                        ]
                else:
                    for t in range(_TD // _L):
                        plane_v[pl.ds(dst + t * _L, _L)] = rvecs[
                            (dt - DHALF) * (_TD // _L) + t
                        ]
            return carry

        HALF = PLANE // 2
        copies = []
        for half in range(2):
            lax.fori_loop(half * (W // 2), (half + 1) * (W // 2), w_body, 0)
            for b in range(B):
                off = pl.multiple_of((b * H + h) * PLANE + half * HALF, HALF)
                cp = pltpu.make_async_copy(
                    plane_v.at[pl.ds(half * HALF, HALF)],
                    out_hbm.at[pl.ds(off, HALF)],
                    sem,
                )
                cp.start()
                copies.append(cp)
        for cp in copies:
            cp.wait()

    return sc_call


def kernel(pixel_values, row_embeddings, column_embeddings):
    B = pixel_values.shape[0]
    H, W = pixel_values.shape[-2], pixel_values.shape[-1]
    D = row_embeddings.shape[-1]
    D2 = 2 * D
    col = (
        column_embeddings[:W]
        .reshape(W // _TW, _TW, D // _TD, _TD)
        .transpose(0, 2, 1, 3)
        .reshape(-1)
    )
    row = (
        row_embeddings[:H]
        .reshape(H // _TW, _TW, D // _TD, _TD)
        .transpose(0, 2, 1, 3)
        .reshape(-1)
    )
    out = _build_sc_call(B, H, W, D)(col, row)
    out6 = out.reshape(B, H, W // _TW, D2 // _TD, _TW, _TD)
    return jnp.transpose(out6, (0, 3, 5, 1, 2, 4)).reshape(B, D2, H, W)
